# baseline (speedup 1.0000x reference)
import functools

import jax
import jax.numpy as jnp
from jax import lax
from jax.experimental import pallas as pl
from jax.experimental.pallas import tpu as pltpu
from jax.experimental.pallas import tpu_sc as plsc

_SIZE = 65536
_DIM = 128
_BATCH = 4096
_NC = 2
_NS = 16
_NW = _NC * _NS
_ROWS = _SIZE // _NW
_WINB = _BATCH // _ROWS
_CR = 256
_NCHUNK = _ROWS // _CR
_NBUF = 3


@functools.cache
def _build():
    @functools.partial(
        pl.kernel,
        out_type=(
            jax.ShapeDtypeStruct((_SIZE, _DIM), jnp.float32),
            jax.ShapeDtypeStruct((16,), jnp.int32),
        ),
        mesh=plsc.VectorSubcoreMesh(
            core_axis_name="c", subcore_axis_name="s",
            num_cores=_NC, num_subcores=_NS,
        ),
        scratch_types=[
            pltpu.VMEM((_NBUF, _CR, _DIM), jnp.float32),
            pltpu.VMEM((16,), jnp.int32),
            pltpu.VMEM((16,), jnp.int32),
            pltpu.SemaphoreType.DMA((_NBUF,)),
            pltpu.SemaphoreType.DMA((_NBUF,)),
        ],
        compiler_params=pltpu.CompilerParams(needs_layout_passes=False),
    )
    def _sc(queue_hbm, feats_hbm, ptr_hbm, out_hbm, nptr_hbm,
            bufs, ptr_v, nptr_v, sem_in, sem_out):
        w = lax.axis_index("s") * _NC + lax.axis_index("c")
        pltpu.sync_copy(ptr_hbm, ptr_v)
        ptr_vec = ptr_v[...]
        p_blk = jnp.max(ptr_vec) // _ROWS
        j = lax.rem(w - p_blk + _NW, _NW)
        base = w * _ROWS

        def start_in(c):
            s = c % _NBUF

            @pl.when(j < _WINB)
            def _():
                pltpu.make_async_copy(
                    feats_hbm.at[pl.ds(j * _ROWS + c * _CR, _CR)],
                    bufs.at[s], sem_in.at[s]).start()

            @pl.when(j >= _WINB)
            def _():
                pltpu.make_async_copy(
                    queue_hbm.at[pl.ds(base + c * _CR, _CR)],
                    bufs.at[s], sem_in.at[s]).start()

        def wait_in(c):
            s = c % _NBUF
            pltpu.make_async_copy(
                queue_hbm.at[pl.ds(base + c * _CR, _CR)],
                bufs.at[s], sem_in.at[s]).wait()

        def start_out(c):
            s = c % _NBUF
            pltpu.make_async_copy(
                bufs.at[s], out_hbm.at[pl.ds(base + c * _CR, _CR)],
                sem_out.at[s]).start()

        def wait_out(c):
            s = c % _NBUF
            pltpu.make_async_copy(
                bufs.at[s], out_hbm.at[pl.ds(base + c * _CR, _CR)],
                sem_out.at[s]).wait()

        for c in range(_NBUF - 1):
            start_in(c)

        @pl.when(w == 0)
        def _():
            nptr_v[...] = lax.rem(ptr_vec + _BATCH, _SIZE)
            pltpu.sync_copy(nptr_v, nptr_hbm)

        for c in range(_NCHUNK):
            wait_in(c)
            start_out(c)
            nxt = c + _NBUF - 1
            if nxt < _NCHUNK:
                if nxt >= _NBUF:
                    wait_out(nxt - _NBUF)
                start_in(nxt)

        for c in range(max(0, _NCHUNK - _NBUF), _NCHUNK):
            wait_out(c)

    return _sc


def kernel(queue, feats, ptr):
    ptr_arr = jnp.full((16,), ptr, dtype=jnp.int32)
    new_queue, nptr16 = _build()(queue, feats, ptr_arr)
    return new_queue, nptr16[:1]


# speedup vs baseline: 1.9598x; 1.9598x over previous
import jax
import jax.numpy as jnp
from jax import lax
from jax.experimental import pallas as pl
from jax.experimental.pallas import tpu as pltpu

_SIZE = 65536
_DIM = 128
_BATCH = 4096
_R = 2048
_NB = _SIZE // _R
_WINB = _BATCH // _R
_NBUF = 16
_PRE = 6
_BLKE = _R * _DIM


def _body(p_ref, q_hbm, f_hbm, o_hbm, np_ref, bufs, sem_in, sem_out):
    p_blk = p_ref[0] // _R

    def start_in(b):
        s = b % _NBUF
        j = lax.rem(b - p_blk + _NB, _NB)

        @pl.when(j < _WINB)
        def _():
            pltpu.make_async_copy(
                f_hbm.at[pl.ds(j * _BLKE, _BLKE)], bufs.at[s], sem_in.at[s]
            ).start()

        @pl.when(j >= _WINB)
        def _():
            pltpu.make_async_copy(
                q_hbm.at[pl.ds(b * _BLKE, _BLKE)], bufs.at[s], sem_in.at[s]
            ).start()

    def wait_in(b):
        s = b % _NBUF
        pltpu.make_async_copy(
            q_hbm.at[pl.ds(b * _BLKE, _BLKE)], bufs.at[s], sem_in.at[s]
        ).wait()

    def start_out(b):
        s = b % _NBUF
        pltpu.make_async_copy(
            bufs.at[s], o_hbm.at[pl.ds(b * _BLKE, _BLKE)], sem_out.at[s]
        ).start()

    def wait_out(b):
        s = b % _NBUF
        pltpu.make_async_copy(
            bufs.at[s], o_hbm.at[pl.ds(b * _BLKE, _BLKE)], sem_out.at[s]
        ).wait()

    for b in range(_PRE):
        start_in(b)

    np_ref[0] = lax.rem(p_ref[0] + _BATCH, _SIZE)

    for b in range(_NB):
        wait_in(b)
        start_out(b)
        nxt = b + _PRE
        if nxt < _NB:
            if nxt >= _NBUF:
                wait_out(nxt - _NBUF)
            start_in(nxt)

    for b in range(max(0, _NB - _NBUF), _NB):
        wait_out(b)


def _run(p_arr, queue_flat, feats_flat):
    grid_spec = pltpu.PrefetchScalarGridSpec(
        num_scalar_prefetch=1,
        grid=(1,),
        in_specs=[
            pl.BlockSpec(memory_space=pl.ANY),
            pl.BlockSpec(memory_space=pl.ANY),
        ],
        out_specs=[
            pl.BlockSpec(memory_space=pl.ANY),
            pl.BlockSpec(memory_space=pltpu.SMEM),
        ],
        scratch_shapes=[
            pltpu.VMEM((_NBUF, _BLKE), jnp.float32),
            pltpu.SemaphoreType.DMA((_NBUF,)),
            pltpu.SemaphoreType.DMA((_NBUF,)),
        ],
    )
    return pl.pallas_call(
        _body,
        grid_spec=grid_spec,
        out_shape=[
            jax.ShapeDtypeStruct((_SIZE * _DIM,), jnp.float32),
            jax.ShapeDtypeStruct((1,), jnp.int32),
        ],
        compiler_params=pltpu.CompilerParams(
            dimension_semantics=("arbitrary",),
        ),
    )(p_arr, queue_flat, feats_flat)


def kernel(queue, feats, ptr):
    p_arr = jnp.reshape(ptr, (1,)).astype(jnp.int32)
    new_queue_flat, new_ptr = _run(p_arr, queue.reshape(-1), feats.reshape(-1))
    return new_queue_flat.reshape(_SIZE, _DIM), new_ptr
